# baseline (speedup 1.0000x reference)
import functools

import jax
import jax.numpy as jnp
from jax import lax
from jax.experimental import pallas as pl
from jax.experimental.pallas import tpu as pltpu
from jax.experimental.pallas import tpu_sc as plsc

B = 16384
E = 64
T = 128
V = 128
H = 128
F = 2 * E + T + V

NC = 2
NS = 16
NW = NC * NS
BPW = B // NW
IDX_ROWS = BPW // 128


def _sc_gather_body(uidx_hbm, ridx_hbm, utab_hbm, rtab_hbm,
                    uout_hbm, rout_hbm,
                    uidx_v, ridx_v, urows_v, rrows_v, sem_u, sem_r):
    wid = lax.axis_index("s") * NC + lax.axis_index("c")
    base = wid * BPW
    row0 = wid * IDX_ROWS
    pltpu.sync_copy(uidx_hbm.at[pl.ds(row0, IDX_ROWS)], uidx_v)
    pltpu.sync_copy(ridx_hbm.at[pl.ds(row0, IDX_ROWS)], ridx_v)
    copies = []
    for j in range(IDX_ROWS):
        copies.append(pltpu.async_copy(
            utab_hbm.at[uidx_v.at[j]], urows_v.at[pl.ds(j * 128, 128)], sem_u))
        copies.append(pltpu.async_copy(
            rtab_hbm.at[ridx_v.at[j]], rrows_v.at[pl.ds(j * 128, 128)], sem_r))
    for c in copies:
        c.wait()
    pltpu.sync_copy(urows_v, uout_hbm.at[pl.ds(base, BPW)])
    pltpu.sync_copy(rrows_v, rout_hbm.at[pl.ds(base, BPW)])


def _make_sc_gather():
    mesh = plsc.VectorSubcoreMesh(core_axis_name="c", subcore_axis_name="s")
    return functools.partial(
        pl.kernel, mesh=mesh,
        compiler_params=pltpu.CompilerParams(use_tc_tiling_on_sc=False),
        out_type=[jax.ShapeDtypeStruct((B, E), jnp.bfloat16),
                  jax.ShapeDtypeStruct((B, E), jnp.bfloat16)],
        scratch_types=[
            pltpu.VMEM((IDX_ROWS, 128), jnp.int32),
            pltpu.VMEM((IDX_ROWS, 128), jnp.int32),
            pltpu.VMEM((BPW, E), jnp.bfloat16),
            pltpu.VMEM((BPW, E), jnp.bfloat16),
            pltpu.SemaphoreType.DMA,
            pltpu.SemaphoreType.DMA,
        ])(_sc_gather_body)


_sc_gather_cached = None


def _sc_gather(*args):
    global _sc_gather_cached
    if _sc_gather_cached is None:
        _sc_gather_cached = _make_sc_gather()
    return _sc_gather_cached(*args)


BLK = 1024


def _mlp_body(u_ref, r_ref, t_ref, v_ref, w1_ref, b1_ref, w2_ref, b2_ref,
              w3_ref, b3_ref, out_ref):
    x = (u_ref[...].astype(jnp.float32) @ w1_ref[0:E, :]
         + r_ref[...].astype(jnp.float32) @ w1_ref[E:2 * E, :]
         + t_ref[...] @ w1_ref[2 * E:2 * E + T, :]
         + v_ref[...] @ w1_ref[2 * E + T:F, :])
    h = jnp.maximum(x + b1_ref[...], 0.0)
    h = jnp.maximum(h @ w2_ref[...] + b2_ref[...], 0.0)
    logit = jnp.sum(h * w3_ref[...], axis=1) + b3_ref[0, 0]
    out_ref[...] = (1.0 / (1.0 + jnp.exp(-logit))).reshape(1, 1, BLK)


def _mlp(u, r, t, v, W1, b1, W2, b2, W3, b3):
    n = B // BLK
    out = pl.pallas_call(
        _mlp_body,
        grid=(n,),
        in_specs=[
            pl.BlockSpec((BLK, E), lambda i: (i, 0)),
            pl.BlockSpec((BLK, E), lambda i: (i, 0)),
            pl.BlockSpec((BLK, T), lambda i: (i, 0)),
            pl.BlockSpec((BLK, V), lambda i: (i, 0)),
            pl.BlockSpec((F, H), lambda i: (0, 0)),
            pl.BlockSpec((1, H), lambda i: (0, 0)),
            pl.BlockSpec((H, H), lambda i: (0, 0)),
            pl.BlockSpec((1, H), lambda i: (0, 0)),
            pl.BlockSpec((1, H), lambda i: (0, 0)),
            pl.BlockSpec(memory_space=pltpu.SMEM),
        ],
        out_specs=pl.BlockSpec((1, 1, BLK), lambda i: (i, 0, 0)),
        out_shape=jax.ShapeDtypeStruct((n, 1, BLK), jnp.float32),
    )(u, r, t, v, W1, b1.reshape(1, H), W2, b2.reshape(1, H),
      W3.reshape(1, H), b3.reshape(1, 1))
    return out.reshape(B)


def kernel(user_indices, reel_indices, text_vectors, vision_vectors,
           user_table, reel_table, W1, b1, W2, b2, W3, b3):
    uidx = user_indices.astype(jnp.int32).reshape(B // 128, 128)
    ridx = reel_indices.astype(jnp.int32).reshape(B // 128, 128)
    utab = user_table.astype(jnp.bfloat16)
    rtab = reel_table.astype(jnp.bfloat16)
    u, r = _sc_gather(uidx, ridx, utab, rtab)
    return _mlp(u, r, text_vectors, vision_vectors, W1, b1, W2, b2, W3, b3)


# speedup vs baseline: 1.2869x; 1.2869x over previous
import functools

import jax
import jax.numpy as jnp
from jax import lax
from jax.experimental import pallas as pl
from jax.experimental.pallas import tpu as pltpu
from jax.experimental.pallas import tpu_sc as plsc

B = 16384
E = 64
T = 128
V = 128
H = 128
F = 2 * E + T + V

NC = 2
NS = 16
NW = NC * NS
BPW = B // NW
IDX_ROWS = BPW // 128


def _sc_gather_body(uidx_hbm, ridx_hbm, utab_hbm, rtab_hbm,
                    uout_hbm, rout_hbm,
                    uidx_v, ridx_v, urows_v, rrows_v, sem_u, sem_r):
    wid = lax.axis_index("s") * NC + lax.axis_index("c")
    base = wid * BPW
    row0 = wid * IDX_ROWS
    pltpu.sync_copy(uidx_hbm.at[pl.ds(row0, IDX_ROWS)], uidx_v)
    pltpu.sync_copy(ridx_hbm.at[pl.ds(row0, IDX_ROWS)], ridx_v)
    copies = []
    for j in range(IDX_ROWS):
        copies.append(pltpu.async_copy(
            utab_hbm.at[uidx_v.at[j]], urows_v.at[pl.ds(j * 128, 128)], sem_u))
        copies.append(pltpu.async_copy(
            rtab_hbm.at[ridx_v.at[j]], rrows_v.at[pl.ds(j * 128, 128)], sem_r))
    for c in copies:
        c.wait()
    pltpu.sync_copy(urows_v, uout_hbm.at[pl.ds(base, BPW)])
    pltpu.sync_copy(rrows_v, rout_hbm.at[pl.ds(base, BPW)])


def _make_sc_gather():
    mesh = plsc.VectorSubcoreMesh(core_axis_name="c", subcore_axis_name="s")
    return functools.partial(
        pl.kernel, mesh=mesh,
        compiler_params=pltpu.CompilerParams(use_tc_tiling_on_sc=False),
        out_type=[jax.ShapeDtypeStruct((B, E), jnp.float32),
                  jax.ShapeDtypeStruct((B, E), jnp.float32)],
        scratch_types=[
            pltpu.VMEM((IDX_ROWS, 128), jnp.int32),
            pltpu.VMEM((IDX_ROWS, 128), jnp.int32),
            pltpu.VMEM((BPW, E), jnp.float32),
            pltpu.VMEM((BPW, E), jnp.float32),
            pltpu.SemaphoreType.DMA,
            pltpu.SemaphoreType.DMA,
        ])(_sc_gather_body)


_sc_gather_cached = None


def _sc_gather(*args):
    global _sc_gather_cached
    if _sc_gather_cached is None:
        _sc_gather_cached = _make_sc_gather()
    return _sc_gather_cached(*args)

BLK = 1024


def _mlp_body(u_ref, r_ref, t_ref, v_ref, w1_ref, b1_ref, w2_ref, b2_ref,
              w3_ref, b3_ref, out_ref):
    x = (u_ref[...] @ w1_ref[0:E, :]
         + r_ref[...] @ w1_ref[E:2 * E, :]
         + t_ref[...] @ w1_ref[2 * E:2 * E + T, :]
         + v_ref[...] @ w1_ref[2 * E + T:F, :])
    h = jnp.maximum(x + b1_ref[...], 0.0)
    h = jnp.maximum(h @ w2_ref[...] + b2_ref[...], 0.0)
    logit = jnp.sum(h * w3_ref[...], axis=1) + b3_ref[0, 0]
    out_ref[...] = (1.0 / (1.0 + jnp.exp(-logit))).reshape(1, 1, BLK)


def _mlp(u, r, t, v, W1, b1, W2, b2, W3, b3):
    n = B // BLK
    out = pl.pallas_call(
        _mlp_body,
        grid=(n,),
        in_specs=[
            pl.BlockSpec((BLK, E), lambda i: (i, 0)),
            pl.BlockSpec((BLK, E), lambda i: (i, 0)),
            pl.BlockSpec((BLK, T), lambda i: (i, 0)),
            pl.BlockSpec((BLK, V), lambda i: (i, 0)),
            pl.BlockSpec((F, H), lambda i: (0, 0)),
            pl.BlockSpec((1, H), lambda i: (0, 0)),
            pl.BlockSpec((H, H), lambda i: (0, 0)),
            pl.BlockSpec((1, H), lambda i: (0, 0)),
            pl.BlockSpec((1, H), lambda i: (0, 0)),
            pl.BlockSpec(memory_space=pltpu.SMEM),
        ],
        out_specs=pl.BlockSpec((1, 1, BLK), lambda i: (i, 0, 0)),
        out_shape=jax.ShapeDtypeStruct((n, 1, BLK), jnp.float32),
    )(u, r, t, v, W1, b1.reshape(1, H), W2, b2.reshape(1, H),
      W3.reshape(1, H), b3.reshape(1, 1))
    return out.reshape(B)


def kernel(user_indices, reel_indices, text_vectors, vision_vectors,
           user_table, reel_table, W1, b1, W2, b2, W3, b3):
    uidx = user_indices.astype(jnp.int32).reshape(B // 128, 128)
    ridx = reel_indices.astype(jnp.int32).reshape(B // 128, 128)
    u, r = _sc_gather(uidx, ridx, user_table, reel_table)
    return _mlp(u, r, text_vectors, vision_vectors, W1, b1, W2, b2, W3, b3)


# speedup vs baseline: 2.6150x; 2.0320x over previous
import functools

import jax
import jax.numpy as jnp
from jax import lax
from jax.experimental import pallas as pl
from jax.experimental.pallas import tpu as pltpu
from jax.experimental.pallas import tpu_sc as plsc

B = 16384
N = 1000000
E = 64
T = 128
V = 128
H = 128
F = 2 * E + T + V

NC = 2
NS = 16
NW = NC * NS
BPW = B // NW
IDX_ROWS = BPW // 128

CB = 2048
NBLK = (N + CB - 1) // CB
N2 = NBLK * CB


def _rne_bf16_high(x_u32):
    return (x_u32 + 0x7FFF + ((x_u32 >> 16) & 1)) & jnp.uint32(0xFFFF0000)


def _xpose_body(u_ref, r_ref, out_ref):
    ub = lax.bitcast_convert_type(u_ref[...], jnp.uint32)
    rb = lax.bitcast_convert_type(r_ref[...], jnp.uint32)
    word = _rne_bf16_high(ub) | (_rne_bf16_high(rb) >> 16)
    wt = word.T
    rows = jnp.concatenate([wt[0:CB // 2], wt[CB // 2:CB]], axis=1)
    out_ref[...] = lax.bitcast_convert_type(rows, jnp.float32)


def _combine_tables(ut, rt):
    out = pl.pallas_call(
        _xpose_body,
        grid=(NBLK,),
        in_specs=[pl.BlockSpec((E, CB), lambda i: (0, i)),
                  pl.BlockSpec((E, CB), lambda i: (0, i))],
        out_specs=pl.BlockSpec((CB // 2, 2 * E), lambda i: (i, 0)),
        out_shape=jax.ShapeDtypeStruct((N2 // 2, 2 * E), jnp.float32),
    )(ut, rt)
    return out.reshape(N2, E)


def _permute_indices(t):
    blk = t & ~(CB - 1)
    k = t & (CB - 1)
    return blk | ((k << 1) & (CB - 1)) | (k >> 10)


def _sc_gather_body(uidx_hbm, ridx_hbm, tab_hbm, uout_hbm, rout_hbm,
                    uidx_v, ridx_v, urows_v, rrows_v, sem_u, sem_r):
    wid = lax.axis_index("s") * NC + lax.axis_index("c")
    base = wid * BPW
    row0 = wid * IDX_ROWS
    pltpu.sync_copy(uidx_hbm.at[pl.ds(row0, IDX_ROWS)], uidx_v)
    pltpu.sync_copy(ridx_hbm.at[pl.ds(row0, IDX_ROWS)], ridx_v)
    copies = []
    for j in range(IDX_ROWS):
        copies.append(pltpu.async_copy(
            tab_hbm.at[uidx_v.at[j]], urows_v.at[pl.ds(j * 128, 128)], sem_u))
        copies.append(pltpu.async_copy(
            tab_hbm.at[ridx_v.at[j]], rrows_v.at[pl.ds(j * 128, 128)], sem_r))
    for c in copies:
        c.wait()
    pltpu.sync_copy(urows_v, uout_hbm.at[pl.ds(base, BPW)])
    pltpu.sync_copy(rrows_v, rout_hbm.at[pl.ds(base, BPW)])


def _make_sc_gather():
    mesh = plsc.VectorSubcoreMesh(core_axis_name="c", subcore_axis_name="s")
    return functools.partial(
        pl.kernel, mesh=mesh,
        compiler_params=pltpu.CompilerParams(use_tc_tiling_on_sc=False),
        out_type=[jax.ShapeDtypeStruct((B, E), jnp.float32),
                  jax.ShapeDtypeStruct((B, E), jnp.float32)],
        scratch_types=[
            pltpu.VMEM((IDX_ROWS, 128), jnp.int32),
            pltpu.VMEM((IDX_ROWS, 128), jnp.int32),
            pltpu.VMEM((BPW, E), jnp.float32),
            pltpu.VMEM((BPW, E), jnp.float32),
            pltpu.SemaphoreType.DMA,
            pltpu.SemaphoreType.DMA,
        ])(_sc_gather_body)


_sc_gather_cached = None


def _sc_gather(*args):
    global _sc_gather_cached
    if _sc_gather_cached is None:
        _sc_gather_cached = _make_sc_gather()
    return _sc_gather_cached(*args)

BLK = 1024


def _mlp_body(u_ref, r_ref, t_ref, v_ref, w1_ref, b1_ref, w2_ref, b2_ref,
              w3_ref, b3_ref, out_ref):
    uw = lax.bitcast_convert_type(u_ref[...], jnp.uint32)
    rw = lax.bitcast_convert_type(r_ref[...], jnp.uint32)
    u = lax.bitcast_convert_type(uw & jnp.uint32(0xFFFF0000), jnp.float32)
    r = lax.bitcast_convert_type(rw << 16, jnp.float32)
    x = (u @ w1_ref[0:E, :]
         + r @ w1_ref[E:2 * E, :]
         + t_ref[...] @ w1_ref[2 * E:2 * E + T, :]
         + v_ref[...] @ w1_ref[2 * E + T:F, :])
    h = jnp.maximum(x + b1_ref[...], 0.0)
    h = jnp.maximum(h @ w2_ref[...] + b2_ref[...], 0.0)
    logit = jnp.sum(h * w3_ref[...], axis=1) + b3_ref[0, 0]
    out_ref[...] = (1.0 / (1.0 + jnp.exp(-logit))).reshape(1, 1, BLK)


def _mlp(u, r, t, v, W1, b1, W2, b2, W3, b3):
    n = B // BLK
    out = pl.pallas_call(
        _mlp_body,
        grid=(n,),
        in_specs=[
            pl.BlockSpec((BLK, E), lambda i: (i, 0)),
            pl.BlockSpec((BLK, E), lambda i: (i, 0)),
            pl.BlockSpec((BLK, T), lambda i: (i, 0)),
            pl.BlockSpec((BLK, V), lambda i: (i, 0)),
            pl.BlockSpec((F, H), lambda i: (0, 0)),
            pl.BlockSpec((1, H), lambda i: (0, 0)),
            pl.BlockSpec((H, H), lambda i: (0, 0)),
            pl.BlockSpec((1, H), lambda i: (0, 0)),
            pl.BlockSpec((1, H), lambda i: (0, 0)),
            pl.BlockSpec(memory_space=pltpu.SMEM),
        ],
        out_specs=pl.BlockSpec((1, 1, BLK), lambda i: (i, 0, 0)),
        out_shape=jax.ShapeDtypeStruct((n, 1, BLK), jnp.float32),
    )(u, r, t, v, W1, b1.reshape(1, H), W2, b2.reshape(1, H),
      W3.reshape(1, H), b3.reshape(1, 1))
    return out.reshape(B)


def kernel(user_indices, reel_indices, text_vectors, vision_vectors,
           user_table, reel_table, W1, b1, W2, b2, W3, b3):
    combo = _combine_tables(user_table.T, reel_table.T)
    uidx = _permute_indices(user_indices.astype(jnp.int32)).reshape(B // 128, 128)
    ridx = _permute_indices(reel_indices.astype(jnp.int32)).reshape(B // 128, 128)
    u, r = _sc_gather(uidx, ridx, combo)
    return _mlp(u, r, text_vectors, vision_vectors, W1, b1, W2, b2, W3, b3)


# speedup vs baseline: 3.3735x; 1.2900x over previous
import functools

import jax
import jax.numpy as jnp
from jax import lax
from jax.experimental import pallas as pl
from jax.experimental.pallas import tpu as pltpu
from jax.experimental.pallas import tpu_sc as plsc

B = 16384
N = 1000000
E = 64
T = 128
V = 128
H = 128
F = 2 * E + T + V

NC = 2
NS = 16
NW = NC * NS
BPW = B // NW
IDX_ROWS = BPW // 128

CB = 4096
NBLK = (N + CB - 1) // CB
N2 = NBLK * CB


def _rne_bf16_high(x_u32):
    return (x_u32 + 0x7FFF + ((x_u32 >> 16) & 1)) & jnp.uint32(0xFFFF0000)


def _xpose_body(u_ref, r_ref, out_ref):
    ub = lax.bitcast_convert_type(u_ref[...], jnp.uint32)
    rb = lax.bitcast_convert_type(r_ref[...], jnp.uint32)
    word = _rne_bf16_high(ub) | (_rne_bf16_high(rb) >> 16)
    wt = word.T
    rows = jnp.concatenate([wt[0:CB // 2], wt[CB // 2:CB]], axis=1)
    out_ref[...] = lax.bitcast_convert_type(rows, jnp.float32)


def _combine_tables(ut, rt):
    out = pl.pallas_call(
        _xpose_body,
        grid=(NBLK,),
        in_specs=[pl.BlockSpec((E, CB), lambda i: (0, i)),
                  pl.BlockSpec((E, CB), lambda i: (0, i))],
        out_specs=pl.BlockSpec((CB // 2, 2 * E), lambda i: (i, 0)),
        out_shape=jax.ShapeDtypeStruct((N2 // 2, 2 * E), jnp.float32),
    )(ut, rt)
    return out.reshape(N2, E)


def _permute_indices(t):
    blk = t & ~(CB - 1)
    k = t & (CB - 1)
    return blk | ((k << 1) & (CB - 1)) | (k >> (CB.bit_length() - 2))


def _sc_gather_body(uidx_hbm, ridx_hbm, tab_hbm, uout_hbm, rout_hbm,
                    uidx_v, ridx_v, urows_v, rrows_v, sem_u, sem_r):
    wid = lax.axis_index("s") * NC + lax.axis_index("c")
    base = wid * BPW
    row0 = wid * IDX_ROWS
    pltpu.sync_copy(uidx_hbm.at[pl.ds(row0, IDX_ROWS)], uidx_v)
    pltpu.sync_copy(ridx_hbm.at[pl.ds(row0, IDX_ROWS)], ridx_v)
    copies = []
    for j in range(IDX_ROWS):
        copies.append(pltpu.async_copy(
            tab_hbm.at[uidx_v.at[j]], urows_v.at[pl.ds(j * 128, 128)], sem_u))
        copies.append(pltpu.async_copy(
            tab_hbm.at[ridx_v.at[j]], rrows_v.at[pl.ds(j * 128, 128)], sem_r))
    for c in copies:
        c.wait()
    pltpu.sync_copy(urows_v, uout_hbm.at[pl.ds(base, BPW)])
    pltpu.sync_copy(rrows_v, rout_hbm.at[pl.ds(base, BPW)])


def _make_sc_gather():
    mesh = plsc.VectorSubcoreMesh(core_axis_name="c", subcore_axis_name="s")
    return functools.partial(
        pl.kernel, mesh=mesh,
        compiler_params=pltpu.CompilerParams(use_tc_tiling_on_sc=False),
        out_type=[jax.ShapeDtypeStruct((B, E), jnp.float32),
                  jax.ShapeDtypeStruct((B, E), jnp.float32)],
        scratch_types=[
            pltpu.VMEM((IDX_ROWS, 128), jnp.int32),
            pltpu.VMEM((IDX_ROWS, 128), jnp.int32),
            pltpu.VMEM((BPW, E), jnp.float32),
            pltpu.VMEM((BPW, E), jnp.float32),
            pltpu.SemaphoreType.DMA,
            pltpu.SemaphoreType.DMA,
        ])(_sc_gather_body)


_sc_gather_cached = None


def _sc_gather(*args):
    global _sc_gather_cached
    if _sc_gather_cached is None:
        _sc_gather_cached = _make_sc_gather()
    return _sc_gather_cached(*args)

BLK = 1024


def _mlp_body(u_ref, r_ref, t_ref, v_ref, w1_ref, b1_ref, w2_ref, b2_ref,
              w3_ref, b3_ref, out_ref):
    uw = lax.bitcast_convert_type(u_ref[...], jnp.uint32)
    rw = lax.bitcast_convert_type(r_ref[...], jnp.uint32)
    u = lax.bitcast_convert_type(uw & jnp.uint32(0xFFFF0000), jnp.float32)
    r = lax.bitcast_convert_type(rw << 16, jnp.float32)
    x = (u @ w1_ref[0:E, :]
         + r @ w1_ref[E:2 * E, :]
         + t_ref[...] @ w1_ref[2 * E:2 * E + T, :]
         + v_ref[...] @ w1_ref[2 * E + T:F, :])
    h = jnp.maximum(x + b1_ref[...], 0.0)
    h = jnp.maximum(h @ w2_ref[...] + b2_ref[...], 0.0)
    logit = jnp.sum(h * w3_ref[...], axis=1) + b3_ref[0, 0]
    out_ref[...] = (1.0 / (1.0 + jnp.exp(-logit))).reshape(1, 1, BLK)


def _mlp(u, r, t, v, W1, b1, W2, b2, W3, b3):
    n = B // BLK
    out = pl.pallas_call(
        _mlp_body,
        grid=(n,),
        in_specs=[
            pl.BlockSpec((BLK, E), lambda i: (i, 0)),
            pl.BlockSpec((BLK, E), lambda i: (i, 0)),
            pl.BlockSpec((BLK, T), lambda i: (i, 0)),
            pl.BlockSpec((BLK, V), lambda i: (i, 0)),
            pl.BlockSpec((F, H), lambda i: (0, 0)),
            pl.BlockSpec((1, H), lambda i: (0, 0)),
            pl.BlockSpec((H, H), lambda i: (0, 0)),
            pl.BlockSpec((1, H), lambda i: (0, 0)),
            pl.BlockSpec((1, H), lambda i: (0, 0)),
            pl.BlockSpec(memory_space=pltpu.SMEM),
        ],
        out_specs=pl.BlockSpec((1, 1, BLK), lambda i: (i, 0, 0)),
        out_shape=jax.ShapeDtypeStruct((n, 1, BLK), jnp.float32),
    )(u, r, t, v, W1, b1.reshape(1, H), W2, b2.reshape(1, H),
      W3.reshape(1, H), b3.reshape(1, 1))
    return out.reshape(B)


def kernel(user_indices, reel_indices, text_vectors, vision_vectors,
           user_table, reel_table, W1, b1, W2, b2, W3, b3):
    combo = _combine_tables(user_table.T, reel_table.T)
    uidx = _permute_indices(user_indices.astype(jnp.int32)).reshape(B // 128, 128)
    ridx = _permute_indices(reel_indices.astype(jnp.int32)).reshape(B // 128, 128)
    u, r = _sc_gather(uidx, ridx, combo)
    return _mlp(u, r, text_vectors, vision_vectors, W1, b1, W2, b2, W3, b3)


# speedup vs baseline: 4.0832x; 1.2104x over previous
import functools

import jax
import jax.numpy as jnp
from jax import lax
from jax.experimental import pallas as pl
from jax.experimental.pallas import tpu as pltpu
from jax.experimental.pallas import tpu_sc as plsc

B = 16384
N = 1000000
E = 64
T = 128
V = 128
H = 128
F = 2 * E + T + V

NC = 2
NS = 16
NW = NC * NS
BPW = B // NW
IDX_ROWS = BPW // 128

CB = 8192
NBLK = (N + CB - 1) // CB
N2 = NBLK * CB


def _rne_bf16_high(x_u32):
    return (x_u32 + 0x7FFF + ((x_u32 >> 16) & 1)) & jnp.uint32(0xFFFF0000)


def _xpose_body(u_ref, r_ref, out_ref):
    ub = lax.bitcast_convert_type(u_ref[...], jnp.uint32)
    rb = lax.bitcast_convert_type(r_ref[...], jnp.uint32)
    word = _rne_bf16_high(ub) | (_rne_bf16_high(rb) >> 16)
    wt = word.T
    rows = jnp.concatenate([wt[0:CB // 2], wt[CB // 2:CB]], axis=1)
    out_ref[...] = lax.bitcast_convert_type(rows, jnp.float32)


def _combine_tables(ut, rt):
    out = pl.pallas_call(
        _xpose_body,
        grid=(NBLK,),
        in_specs=[pl.BlockSpec((E, CB), lambda i: (0, i)),
                  pl.BlockSpec((E, CB), lambda i: (0, i))],
        out_specs=pl.BlockSpec((CB // 2, 2 * E), lambda i: (i, 0)),
        out_shape=jax.ShapeDtypeStruct((N2 // 2, 2 * E), jnp.float32),
    )(ut, rt)
    return out.reshape(N2, E)


def _permute_indices(t):
    blk = t & ~(CB - 1)
    k = t & (CB - 1)
    return blk | ((k << 1) & (CB - 1)) | (k >> (CB.bit_length() - 2))


def _sc_gather_body(uidx_hbm, ridx_hbm, tab_hbm, uout_hbm, rout_hbm,
                    uidx_v, ridx_v, urows_v, rrows_v, sem_u, sem_r):
    wid = lax.axis_index("s") * NC + lax.axis_index("c")
    base = wid * BPW
    row0 = wid * IDX_ROWS
    pltpu.sync_copy(uidx_hbm.at[pl.ds(row0, IDX_ROWS)], uidx_v)
    pltpu.sync_copy(ridx_hbm.at[pl.ds(row0, IDX_ROWS)], ridx_v)
    copies = []
    for j in range(IDX_ROWS):
        copies.append(pltpu.async_copy(
            tab_hbm.at[uidx_v.at[j]], urows_v.at[pl.ds(j * 128, 128)], sem_u))
        copies.append(pltpu.async_copy(
            tab_hbm.at[ridx_v.at[j]], rrows_v.at[pl.ds(j * 128, 128)], sem_r))
    for c in copies:
        c.wait()
    pltpu.sync_copy(urows_v, uout_hbm.at[pl.ds(base, BPW)])
    pltpu.sync_copy(rrows_v, rout_hbm.at[pl.ds(base, BPW)])


def _make_sc_gather():
    mesh = plsc.VectorSubcoreMesh(core_axis_name="c", subcore_axis_name="s")
    return functools.partial(
        pl.kernel, mesh=mesh,
        compiler_params=pltpu.CompilerParams(use_tc_tiling_on_sc=False),
        out_type=[jax.ShapeDtypeStruct((B, E), jnp.float32),
                  jax.ShapeDtypeStruct((B, E), jnp.float32)],
        scratch_types=[
            pltpu.VMEM((IDX_ROWS, 128), jnp.int32),
            pltpu.VMEM((IDX_ROWS, 128), jnp.int32),
            pltpu.VMEM((BPW, E), jnp.float32),
            pltpu.VMEM((BPW, E), jnp.float32),
            pltpu.SemaphoreType.DMA,
            pltpu.SemaphoreType.DMA,
        ])(_sc_gather_body)


_sc_gather_cached = None


def _sc_gather(*args):
    global _sc_gather_cached
    if _sc_gather_cached is None:
        _sc_gather_cached = _make_sc_gather()
    return _sc_gather_cached(*args)

BLK = 1024


def _mlp_body(u_ref, r_ref, t_ref, v_ref, w1_ref, b1_ref, w2_ref, b2_ref,
              w3_ref, b3_ref, out_ref):
    uw = lax.bitcast_convert_type(u_ref[...], jnp.uint32)
    rw = lax.bitcast_convert_type(r_ref[...], jnp.uint32)
    u = lax.bitcast_convert_type(uw & jnp.uint32(0xFFFF0000), jnp.float32)
    r = lax.bitcast_convert_type(rw << 16, jnp.float32)
    x = (u @ w1_ref[0:E, :]
         + r @ w1_ref[E:2 * E, :]
         + t_ref[...] @ w1_ref[2 * E:2 * E + T, :]
         + v_ref[...] @ w1_ref[2 * E + T:F, :])
    h = jnp.maximum(x + b1_ref[...], 0.0)
    h = jnp.maximum(h @ w2_ref[...] + b2_ref[...], 0.0)
    logit = jnp.sum(h * w3_ref[...], axis=1) + b3_ref[0, 0]
    out_ref[...] = (1.0 / (1.0 + jnp.exp(-logit))).reshape(1, 1, BLK)


def _mlp(u, r, t, v, W1, b1, W2, b2, W3, b3):
    n = B // BLK
    out = pl.pallas_call(
        _mlp_body,
        grid=(n,),
        in_specs=[
            pl.BlockSpec((BLK, E), lambda i: (i, 0)),
            pl.BlockSpec((BLK, E), lambda i: (i, 0)),
            pl.BlockSpec((BLK, T), lambda i: (i, 0)),
            pl.BlockSpec((BLK, V), lambda i: (i, 0)),
            pl.BlockSpec((F, H), lambda i: (0, 0)),
            pl.BlockSpec((1, H), lambda i: (0, 0)),
            pl.BlockSpec((H, H), lambda i: (0, 0)),
            pl.BlockSpec((1, H), lambda i: (0, 0)),
            pl.BlockSpec((1, H), lambda i: (0, 0)),
            pl.BlockSpec(memory_space=pltpu.SMEM),
        ],
        out_specs=pl.BlockSpec((1, 1, BLK), lambda i: (i, 0, 0)),
        out_shape=jax.ShapeDtypeStruct((n, 1, BLK), jnp.float32),
    )(u, r, t, v, W1, b1.reshape(1, H), W2, b2.reshape(1, H),
      W3.reshape(1, H), b3.reshape(1, 1))
    return out.reshape(B)


def kernel(user_indices, reel_indices, text_vectors, vision_vectors,
           user_table, reel_table, W1, b1, W2, b2, W3, b3):
    combo = _combine_tables(user_table.T, reel_table.T)
    uidx = _permute_indices(user_indices.astype(jnp.int32)).reshape(B // 128, 128)
    ridx = _permute_indices(reel_indices.astype(jnp.int32)).reshape(B // 128, 128)
    u, r = _sc_gather(uidx, ridx, combo)
    return _mlp(u, r, text_vectors, vision_vectors, W1, b1, W2, b2, W3, b3)


# speedup vs baseline: 4.5395x; 1.1118x over previous
import functools

import jax
import jax.numpy as jnp
from jax import lax
from jax.experimental import pallas as pl
from jax.experimental.pallas import tpu as pltpu
from jax.experimental.pallas import tpu_sc as plsc

B = 16384
N = 1000000
E = 64
T = 128
V = 128
H = 128
F = 2 * E + T + V

NC = 2
NS = 16
NW = NC * NS
BPW = B // NW
IDX_ROWS = BPW // 128

CB = 16384
NBLK = (N + CB - 1) // CB
N2 = NBLK * CB


def _rne_bf16_high(x_u32):
    return (x_u32 + 0x7FFF + ((x_u32 >> 16) & 1)) & jnp.uint32(0xFFFF0000)


def _xpose_body(u_ref, r_ref, out_ref):
    ub = lax.bitcast_convert_type(u_ref[...], jnp.uint32)
    rb = lax.bitcast_convert_type(r_ref[...], jnp.uint32)
    word = _rne_bf16_high(ub) | (_rne_bf16_high(rb) >> 16)
    wt = word.T
    rows = jnp.concatenate([wt[0:CB // 2], wt[CB // 2:CB]], axis=1)
    out_ref[...] = lax.bitcast_convert_type(rows, jnp.float32)


def _combine_tables(ut, rt):
    out = pl.pallas_call(
        _xpose_body,
        grid=(NBLK,),
        in_specs=[pl.BlockSpec((E, CB), lambda i: (0, i)),
                  pl.BlockSpec((E, CB), lambda i: (0, i))],
        out_specs=pl.BlockSpec((CB // 2, 2 * E), lambda i: (i, 0)),
        out_shape=jax.ShapeDtypeStruct((N2 // 2, 2 * E), jnp.float32),
    )(ut, rt)
    return out.reshape(N2, E)


def _permute_indices(t):
    blk = t & ~(CB - 1)
    k = t & (CB - 1)
    return blk | ((k << 1) & (CB - 1)) | (k >> (CB.bit_length() - 2))


def _sc_gather_body(uidx_hbm, ridx_hbm, tab_hbm, uout_hbm, rout_hbm,
                    uidx_v, ridx_v, urows_v, rrows_v, sem_u, sem_r):
    wid = lax.axis_index("s") * NC + lax.axis_index("c")
    base = wid * BPW
    row0 = wid * IDX_ROWS
    pltpu.sync_copy(uidx_hbm.at[pl.ds(row0, IDX_ROWS)], uidx_v)
    pltpu.sync_copy(ridx_hbm.at[pl.ds(row0, IDX_ROWS)], ridx_v)
    copies = []
    for j in range(IDX_ROWS):
        copies.append(pltpu.async_copy(
            tab_hbm.at[uidx_v.at[j]], urows_v.at[pl.ds(j * 128, 128)], sem_u))
        copies.append(pltpu.async_copy(
            tab_hbm.at[ridx_v.at[j]], rrows_v.at[pl.ds(j * 128, 128)], sem_r))
    for c in copies:
        c.wait()
    pltpu.sync_copy(urows_v, uout_hbm.at[pl.ds(base, BPW)])
    pltpu.sync_copy(rrows_v, rout_hbm.at[pl.ds(base, BPW)])


def _make_sc_gather():
    mesh = plsc.VectorSubcoreMesh(core_axis_name="c", subcore_axis_name="s")
    return functools.partial(
        pl.kernel, mesh=mesh,
        compiler_params=pltpu.CompilerParams(use_tc_tiling_on_sc=False),
        out_type=[jax.ShapeDtypeStruct((B, E), jnp.float32),
                  jax.ShapeDtypeStruct((B, E), jnp.float32)],
        scratch_types=[
            pltpu.VMEM((IDX_ROWS, 128), jnp.int32),
            pltpu.VMEM((IDX_ROWS, 128), jnp.int32),
            pltpu.VMEM((BPW, E), jnp.float32),
            pltpu.VMEM((BPW, E), jnp.float32),
            pltpu.SemaphoreType.DMA,
            pltpu.SemaphoreType.DMA,
        ])(_sc_gather_body)


_sc_gather_cached = None


def _sc_gather(*args):
    global _sc_gather_cached
    if _sc_gather_cached is None:
        _sc_gather_cached = _make_sc_gather()
    return _sc_gather_cached(*args)

BLK = 1024


def _mlp_body(u_ref, r_ref, t_ref, v_ref, w1_ref, b1_ref, w2_ref, b2_ref,
              w3_ref, b3_ref, out_ref):
    uw = lax.bitcast_convert_type(u_ref[...], jnp.uint32)
    rw = lax.bitcast_convert_type(r_ref[...], jnp.uint32)
    u = lax.bitcast_convert_type(uw & jnp.uint32(0xFFFF0000), jnp.float32)
    r = lax.bitcast_convert_type(rw << 16, jnp.float32)
    x = (u @ w1_ref[0:E, :]
         + r @ w1_ref[E:2 * E, :]
         + t_ref[...] @ w1_ref[2 * E:2 * E + T, :]
         + v_ref[...] @ w1_ref[2 * E + T:F, :])
    h = jnp.maximum(x + b1_ref[...], 0.0)
    h = jnp.maximum(h @ w2_ref[...] + b2_ref[...], 0.0)
    logit = jnp.sum(h * w3_ref[...], axis=1) + b3_ref[0, 0]
    out_ref[...] = (1.0 / (1.0 + jnp.exp(-logit))).reshape(1, 1, BLK)


def _mlp(u, r, t, v, W1, b1, W2, b2, W3, b3):
    n = B // BLK
    out = pl.pallas_call(
        _mlp_body,
        grid=(n,),
        in_specs=[
            pl.BlockSpec((BLK, E), lambda i: (i, 0)),
            pl.BlockSpec((BLK, E), lambda i: (i, 0)),
            pl.BlockSpec((BLK, T), lambda i: (i, 0)),
            pl.BlockSpec((BLK, V), lambda i: (i, 0)),
            pl.BlockSpec((F, H), lambda i: (0, 0)),
            pl.BlockSpec((1, H), lambda i: (0, 0)),
            pl.BlockSpec((H, H), lambda i: (0, 0)),
            pl.BlockSpec((1, H), lambda i: (0, 0)),
            pl.BlockSpec((1, H), lambda i: (0, 0)),
            pl.BlockSpec(memory_space=pltpu.SMEM),
        ],
        out_specs=pl.BlockSpec((1, 1, BLK), lambda i: (i, 0, 0)),
        out_shape=jax.ShapeDtypeStruct((n, 1, BLK), jnp.float32),
    )(u, r, t, v, W1, b1.reshape(1, H), W2, b2.reshape(1, H),
      W3.reshape(1, H), b3.reshape(1, 1))
    return out.reshape(B)


def kernel(user_indices, reel_indices, text_vectors, vision_vectors,
           user_table, reel_table, W1, b1, W2, b2, W3, b3):
    combo = _combine_tables(user_table.T, reel_table.T)
    uidx = _permute_indices(user_indices.astype(jnp.int32)).reshape(B // 128, 128)
    ridx = _permute_indices(reel_indices.astype(jnp.int32)).reshape(B // 128, 128)
    u, r = _sc_gather(uidx, ridx, combo)
    return _mlp(u, r, text_vectors, vision_vectors, W1, b1, W2, b2, W3, b3)
